# baseline (speedup 1.0000x reference)
import jax
import jax.numpy as jnp
from jax.experimental import pallas as pl
from jax.experimental.pallas import tpu as pltpu

_BLOCK_T = 256
_N_BUFS = 9


def _router_ring(x_hbm, wt_ref, b_ref, o_ref, xbuf, sems):
    n_tokens, d_model = x_hbm.shape
    n_blocks = n_tokens // _BLOCK_T

    def fetch(blk, slot):
        pltpu.make_async_copy(
            x_hbm.at[pl.ds(blk * _BLOCK_T, _BLOCK_T), :],
            xbuf.at[slot],
            sems.at[slot],
        ).start()

    for k in range(_N_BUFS):
        fetch(k, k)

    bias = b_ref[...]
    for i in range(n_blocks):
        s = i % _N_BUFS
        pltpu.make_async_copy(
            x_hbm.at[pl.ds(i * _BLOCK_T, _BLOCK_T), :],
            xbuf.at[s],
            sems.at[s],
        ).wait()
        logits = jnp.dot(xbuf[s], wt_ref[...],
                         preferred_element_type=jnp.float32) + bias
        m = jnp.max(logits, axis=-1, keepdims=True)
        e = jnp.exp(logits - m)
        o_ref[pl.ds(i * _BLOCK_T, _BLOCK_T), :] = (
            e / jnp.sum(e, axis=-1, keepdims=True))
        if i + _N_BUFS < n_blocks:
            fetch(i + _N_BUFS, s)


def kernel(x, W, b):
    n_tokens, d_model = x.shape
    n_experts = W.shape[0]
    wt = W.T
    b2 = b.reshape(1, n_experts)
    return pl.pallas_call(
        _router_ring,
        in_specs=[
            pl.BlockSpec(memory_space=pltpu.MemorySpace.HBM),
            pl.BlockSpec((d_model, n_experts), lambda: (0, 0)),
            pl.BlockSpec((1, n_experts), lambda: (0, 0)),
        ],
        out_specs=pl.BlockSpec((n_tokens, n_experts), lambda: (0, 0)),
        out_shape=jax.ShapeDtypeStruct((n_tokens, n_experts), jnp.float32),
        scratch_shapes=[
            pltpu.VMEM((_N_BUFS, _BLOCK_T, d_model), jnp.float32),
            pltpu.SemaphoreType.DMA((_N_BUFS,)),
        ],
    )(x, wt, b2)


# speedup vs baseline: 1.1182x; 1.1182x over previous
import jax
import jax.numpy as jnp
from jax.experimental import pallas as pl
from jax.experimental.pallas import tpu as pltpu

_BLOCK_T = 256
_N_BUFS = 9


def _router_ring(x_hbm, wt_ref, b_ref, o_ref, xbuf, sems):
    n_tokens, d_model = x_hbm.shape
    n_blocks = n_tokens // _BLOCK_T

    def fetch(blk, slot):
        pltpu.make_async_copy(
            x_hbm.at[pl.ds(blk * _BLOCK_T, _BLOCK_T), :],
            xbuf.at[slot],
            sems.at[slot],
        ).start()

    for k in range(_N_BUFS):
        fetch(k, k)

    bias = b_ref[...]

    def step(i, slot):
        pltpu.make_async_copy(
            x_hbm.at[pl.ds(i * _BLOCK_T, _BLOCK_T), :],
            xbuf.at[slot],
            sems.at[slot],
        ).wait()
        logits = jnp.dot(xbuf[slot], wt_ref[...],
                         preferred_element_type=jnp.float32) + bias
        m = jnp.max(logits, axis=-1, keepdims=True)
        e = jnp.exp(logits - m)
        o_ref[pl.ds(i * _BLOCK_T, _BLOCK_T), :] = (
            e / jnp.sum(e, axis=-1, keepdims=True))

        @pl.when(i + _N_BUFS < n_blocks)
        def _():
            fetch(i + _N_BUFS, slot)

        return jnp.where(slot == _N_BUFS - 1, 0, slot + 1)

    jax.lax.fori_loop(0, n_blocks, step, 0, unroll=False)


def kernel(x, W, b):
    n_tokens, d_model = x.shape
    n_experts = W.shape[0]
    wt = W.T
    b2 = b.reshape(1, n_experts)
    return pl.pallas_call(
        _router_ring,
        in_specs=[
            pl.BlockSpec(memory_space=pltpu.MemorySpace.HBM),
            pl.BlockSpec((d_model, n_experts), lambda: (0, 0)),
            pl.BlockSpec((1, n_experts), lambda: (0, 0)),
        ],
        out_specs=pl.BlockSpec((n_tokens, n_experts), lambda: (0, 0)),
        out_shape=jax.ShapeDtypeStruct((n_tokens, n_experts), jnp.float32),
        scratch_shapes=[
            pltpu.VMEM((_N_BUFS, _BLOCK_T, d_model), jnp.float32),
            pltpu.SemaphoreType.DMA((_N_BUFS,)),
        ],
    )(x, wt, b2)


# speedup vs baseline: 1.1463x; 1.0251x over previous
import jax
import jax.numpy as jnp
from jax.experimental import pallas as pl


def _router_block(x0_ref, x1_ref, wt_ref, b_ref, o_ref):
    half = x0_ref.shape[0]
    bias = b_ref[...]
    for q, x_ref in enumerate((x0_ref, x1_ref)):
        logits = jnp.dot(x_ref[...], wt_ref[...],
                         preferred_element_type=jnp.float32) + bias
        m = jnp.max(logits, axis=-1, keepdims=True)
        e = jnp.exp(logits - m)
        o_ref[pl.ds(q * half, half), :] = (
            e / jnp.sum(e, axis=-1, keepdims=True))


def kernel(x, W, b):
    n_tokens, d_model = x.shape
    n_experts = W.shape[0]
    block_t = 1024
    half = block_t // 2
    wt = W.T
    b2 = b.reshape(1, n_experts)
    return pl.pallas_call(
        _router_block,
        grid=(n_tokens // block_t,),
        in_specs=[
            pl.BlockSpec((half, d_model), lambda i: (2 * i, 0)),
            pl.BlockSpec((half, d_model), lambda i: (2 * i + 1, 0)),
            pl.BlockSpec((d_model, n_experts), lambda i: (0, 0)),
            pl.BlockSpec((1, n_experts), lambda i: (0, 0)),
        ],
        out_specs=pl.BlockSpec((block_t, n_experts), lambda i: (i, 0)),
        out_shape=jax.ShapeDtypeStruct((n_tokens, n_experts), jnp.float32),
    )(x, x, wt, b2)
